# Initial kernel scaffold; baseline (speedup 1.0000x reference)
import functools

import jax
import jax.numpy as jnp
from jax import lax
from jax.experimental import pallas as pl
from jax.experimental.pallas import tpu as pltpu
from jax.experimental.pallas import tpu_sc as plsc

N = 100000
D = 128
S = 256
L = 16
NC = 2
NS = 16
NW = NC * NS
SEG_PER_W = S // NW
R = 64
KV = D // L


def _body(x_hbm, batch_hbm, wb_hbm, out_hbm, batch_v, xbuf_v, wb_v, lob_v, stage_v):
    wid = lax.axis_index("c") * NS + lax.axis_index("s")

    pltpu.sync_copy(wb_hbm, wb_v)
    pltpu.sync_copy(batch_hbm, batch_v)

    w = [wb_v[0, pl.ds(k * L, L)] for k in range(KV)]
    bvec = wb_v[1, pl.ds(0, L)]

    t = wid * SEG_PER_W + lax.iota(jnp.int32, L)
    lo0 = jnp.zeros((L,), jnp.int32)
    hi0 = jnp.full((L,), N, jnp.int32)

    def sbody(_, c):
        lo, hi = c
        act = lo < hi
        mid = lax.shift_right_logical(lo + hi, 1)
        vals = plsc.load_gather(batch_v, [jnp.minimum(mid, N - 1)])
        less = vals < t
        lo = jnp.where(act & less, mid + 1, lo)
        hi = jnp.where(act & (~less), mid, hi)
        return lo, hi

    lo, _ = lax.fori_loop(0, 17, sbody, (lo0, hi0))
    lob_v[...] = lo

    zero = jnp.zeros((L,), jnp.float32)
    ninf = jnp.full((L,), -jnp.inf, jnp.float32)

    for j in range(SEG_PER_W):
        seg_lo = lob_v[j]
        seg_hi = lob_v[j + 1]

        def cond_fn(c):
            return c[0] < seg_hi

        def chunk(c):
            r = c[0]
            dstart = jnp.minimum(r, N - R)
            pltpu.sync_copy(x_hbm.at[pl.ds(dstart, R)], xbuf_v)
            off = r - dstart
            nrows = jnp.minimum(R - off, seg_hi - r)

            def row(i, c2):
                ri = off + i
                xs = [xbuf_v[ri, pl.ds(k * L, L)] for k in range(KV)]
                acc = xs[0] * w[0]
                for k in range(1, KV):
                    acc = acc + xs[k] * w[k]
                d = jnp.sum(acc)
                zv = jnp.full((L,), d, jnp.float32) + bvec
                sv = 1.0 / (1.0 + jnp.exp(-zv))
                sums = tuple(c2[k] + sv * xs[k] for k in range(KV))
                maxs = tuple(jnp.maximum(c2[KV + k], xs[k]) for k in range(KV))
                return sums + maxs

            res = lax.fori_loop(0, nrows, row, c[1:])
            return (r + nrows,) + res

        fin = lax.while_loop(cond_fn, chunk, (seg_lo,) + (zero,) * KV + (ninf,) * KV)
        for k in range(KV):
            stage_v[j, pl.ds(k * L, L)] = fin[1 + k]
            stage_v[j, pl.ds(D + k * L, L)] = fin[1 + KV + k]

    pltpu.sync_copy(stage_v, out_hbm.at[pl.ds(wid * SEG_PER_W, SEG_PER_W)])


@jax.jit
def kernel(x, batch, W, b):
    batch32 = batch.astype(jnp.int32)
    wb = jnp.concatenate(
        [W.astype(jnp.float32),
         jnp.broadcast_to(b.astype(jnp.float32).reshape(1, 1), (1, D))], axis=0)
    mesh = plsc.VectorSubcoreMesh(core_axis_name="c", subcore_axis_name="s")
    fn = functools.partial(
        pl.kernel,
        out_type=jax.ShapeDtypeStruct((S, 2 * D), jnp.float32),
        mesh=mesh,
        scratch_types=[
            pltpu.VMEM((N,), jnp.int32),
            pltpu.VMEM((R, D), jnp.float32),
            pltpu.VMEM((2, D), jnp.float32),
            pltpu.VMEM((L,), jnp.int32),
            pltpu.VMEM((SEG_PER_W, 2 * D), jnp.float32),
        ],
    )(_body)
    return fn(x, batch32, wb)


# speedup vs baseline: 5.4507x; 5.4507x over previous
import functools

import jax
import jax.numpy as jnp
from jax import lax
from jax.experimental import pallas as pl
from jax.experimental.pallas import tpu as pltpu
from jax.experimental.pallas import tpu_sc as plsc

N = 100000
D = 128
S = 256
L = 16
NC = 2
NS = 16
NW = NC * NS
SEG_PER_W = S // NW
R = 64
KV = D // L


def _body(x_hbm, batch_hbm, wb_hbm, out_hbm, batch_v, xbuf_v, wb_v, stage_v):
    wid = lax.axis_index("c") * NS + lax.axis_index("s")

    pltpu.sync_copy(wb_hbm, wb_v)
    pltpu.sync_copy(batch_hbm, batch_v)

    w = [wb_v[0, pl.ds(k * L, L)] for k in range(KV)]
    bvec = wb_v[1, pl.ds(0, L)]

    t = wid * SEG_PER_W + lax.iota(jnp.int32, L)
    lo0 = jnp.zeros((L,), jnp.int32)
    hi0 = jnp.full((L,), N, jnp.int32)

    def sbody(_, c):
        lo, hi = c
        act = lo < hi
        mid = lax.shift_right_logical(lo + hi, 1)
        vals = plsc.load_gather(batch_v, [jnp.minimum(mid, N - 1)])
        less = vals < t
        lo = jnp.where(act & less, mid + 1, lo)
        hi = jnp.where(act & (~less), mid, hi)
        return lo, hi

    lo, _ = lax.fori_loop(0, 17, sbody, (lo0, hi0))

    zero = jnp.zeros((L,), jnp.float32)
    ninf = jnp.full((L,), -jnp.inf, jnp.float32)

    for j in range(SEG_PER_W):
        seg_lo = lo[j]
        seg_hi = lo[j + 1]

        def cond_fn(c):
            return c[0] < seg_hi

        def chunk(c):
            r = c[0]
            dstart = pl.multiple_of(jnp.minimum(r & ~7, N - R), 8)
            pltpu.sync_copy(x_hbm.at[pl.ds(dstart, R)], xbuf_v)
            off = r - dstart
            nrows = jnp.minimum(R - off, seg_hi - r)

            def row(i, c2):
                ri = off + i
                xs = [xbuf_v[ri, pl.ds(k * L, L)] for k in range(KV)]
                acc = xs[0] * w[0]
                for k in range(1, KV):
                    acc = acc + xs[k] * w[k]
                d = jnp.sum(acc)
                zv = jnp.full((L,), d, jnp.float32) + bvec
                sv = 1.0 / (1.0 + jnp.exp(-zv))
                sums = tuple(c2[k] + sv * xs[k] for k in range(KV))
                maxs = tuple(jnp.maximum(c2[KV + k], xs[k]) for k in range(KV))
                return sums + maxs

            res = lax.fori_loop(0, nrows, row, c[1:])
            return (r + nrows,) + res

        fin = lax.while_loop(cond_fn, chunk, (seg_lo,) + (zero,) * KV + (ninf,) * KV)
        for k in range(KV):
            stage_v[j, pl.ds(k * L, L)] = fin[1 + k]
            stage_v[j, pl.ds(D + k * L, L)] = fin[1 + KV + k]

    pltpu.sync_copy(stage_v, out_hbm.at[pl.ds(wid * SEG_PER_W, SEG_PER_W)])


@jax.jit
def kernel(x, batch, W, b):
    batch32 = batch.astype(jnp.int32)
    wb = jnp.concatenate(
        [W.astype(jnp.float32),
         jnp.broadcast_to(b.astype(jnp.float32).reshape(1, 1), (1, D))], axis=0)
    mesh = plsc.VectorSubcoreMesh(core_axis_name="c", subcore_axis_name="s")
    fn = functools.partial(
        pl.kernel,
        out_type=jax.ShapeDtypeStruct((S, 2 * D), jnp.float32),
        mesh=mesh,
        compiler_params=pltpu.CompilerParams(needs_layout_passes=False),
        scratch_types=[
            pltpu.VMEM((N,), jnp.int32),
            pltpu.VMEM((R, D), jnp.float32),
            pltpu.VMEM((2, D), jnp.float32),
            pltpu.VMEM((SEG_PER_W, 2 * D), jnp.float32),
        ],
    )(_body)
    return fn(x, batch32, wb)
